# baseline (speedup 1.0000x reference)
import jax
import jax.numpy as jnp
from jax import lax
from jax.experimental import pallas as pl
from jax.experimental.pallas import tpu as pltpu

_BATCH = 4096
_NCARDS = 26
_CARD = 100
_WIDTH = _NCARDS * _CARD
_MAIN = 2560
_TAIL = _WIDTH - _MAIN
_BR = 512
_NBLK = _BATCH // _BR


def _copies(main_s, tail_s, o_ref, buf, blk, sem_a, sem_b):
    rows = pl.ds(blk * _BR, _BR)
    main = pltpu.make_async_copy(
        main_s.at[buf], o_ref.at[rows, pl.ds(0, _MAIN)], sem_a.at[buf])
    tail = pltpu.make_async_copy(
        tail_s.at[buf], o_ref.at[rows, pl.ds(_MAIN, _TAIL)], sem_b.at[buf])
    return main, tail


def _onehot(x_ref, rm_ref, rt_ref, o_ref, main_s, tail_s, sem_a, sem_b):
    jm = lax.broadcasted_iota(jnp.int32, (_BR, _MAIN), 1)
    pos_m = (jm - (jm // _CARD) * _CARD).astype(jnp.float32)
    jt = lax.broadcasted_iota(jnp.int32, (_BR, _TAIL), 1) + _MAIN
    pos_t = (jt - (jt // _CARD) * _CARD).astype(jnp.float32)
    for blk in range(_NBLK):
        buf = blk % 2
        if blk >= 2:
            pm, pt = _copies(main_s, tail_s, o_ref, buf, blk - 2, sem_a, sem_b)
            pm.wait()
            pt.wait()
        xb = x_ref[pl.ds(blk * _BR, _BR), :]
        xr_m = jnp.dot(xb, rm_ref[...], preferred_element_type=jnp.float32)
        xr_t = jnp.dot(xb, rt_ref[...], preferred_element_type=jnp.float32)
        main_s[buf] = (xr_m == pos_m).astype(jnp.int32)
        tail_s[buf] = (xr_t == pos_t).astype(jnp.int32)
        m, t = _copies(main_s, tail_s, o_ref, buf, blk, sem_a, sem_b)
        m.start()
        t.start()
    for blk in range(_NBLK - 2, _NBLK):
        pm, pt = _copies(main_s, tail_s, o_ref, blk % 2, blk, sem_a, sem_b)
        pm.wait()
        pt.wait()


def kernel(x):
    xb = x.astype(jnp.bfloat16)
    card_of_col = jnp.arange(_WIDTH, dtype=jnp.int32) // _CARD
    rep = (card_of_col[None, :] == jnp.arange(_NCARDS, dtype=jnp.int32)[:, None]
           ).astype(jnp.bfloat16)
    return pl.pallas_call(
        _onehot,
        in_specs=[
            pl.BlockSpec(memory_space=pltpu.VMEM),
            pl.BlockSpec(memory_space=pltpu.VMEM),
            pl.BlockSpec(memory_space=pltpu.VMEM),
        ],
        out_specs=pl.BlockSpec(memory_space=pl.ANY),
        out_shape=jax.ShapeDtypeStruct((_BATCH, _WIDTH), jnp.int32),
        scratch_shapes=[
            pltpu.VMEM((2, _BR, _MAIN), jnp.int32),
            pltpu.VMEM((2, _BR, _TAIL), jnp.int32),
            pltpu.SemaphoreType.DMA((2,)),
            pltpu.SemaphoreType.DMA((2,)),
        ],
    )(xb, rep[:, :_MAIN], rep[:, _MAIN:])


# speedup vs baseline: 1.0093x; 1.0093x over previous
import jax
import jax.numpy as jnp
from jax import lax
from jax.experimental import pallas as pl
from jax.experimental.pallas import tpu as pltpu

_BATCH = 4096
_NCARDS = 26
_CARD = 100
_WIDTH = _NCARDS * _CARD
_PADW = 2688
_BR = 512
_NBLK = _BATCH // _BR


def _copy(scratch, o_ref, buf, blk, sem):
    rows = pl.ds(blk * _BR, _BR)
    return pltpu.make_async_copy(
        scratch.at[buf], o_ref.at[rows, pl.ds(0, _PADW)], sem.at[buf])


def _onehot(x_ref, r_ref, o_ref, scratch, sem):
    j = lax.broadcasted_iota(jnp.int32, (_BR, _PADW), 1)
    pos = (j - (j // _CARD) * _CARD).astype(jnp.float32)
    for blk in range(_NBLK):
        buf = blk % 2
        if blk >= 2:
            _copy(scratch, o_ref, buf, blk - 2, sem).wait()
        xb = x_ref[pl.ds(blk * _BR, _BR), :]
        xr = jnp.dot(xb, r_ref[...], preferred_element_type=jnp.float32)
        scratch[buf] = (xr == pos).astype(jnp.int32)
        _copy(scratch, o_ref, buf, blk, sem).start()
    for blk in range(_NBLK - 2, _NBLK):
        _copy(scratch, o_ref, blk % 2, blk, sem).wait()


def kernel(x):
    xb = x.astype(jnp.bfloat16)
    col = jnp.arange(_PADW, dtype=jnp.int32)
    card_of_col = col // _CARD
    rep = (card_of_col[None, :] == jnp.arange(_NCARDS, dtype=jnp.int32)[:, None]
           ).astype(jnp.bfloat16)
    return pl.pallas_call(
        _onehot,
        in_specs=[
            pl.BlockSpec(memory_space=pltpu.VMEM),
            pl.BlockSpec(memory_space=pltpu.VMEM),
        ],
        out_specs=pl.BlockSpec(memory_space=pl.ANY),
        out_shape=jax.ShapeDtypeStruct((_BATCH, _WIDTH), jnp.int32),
        scratch_shapes=[
            pltpu.VMEM((2, _BR, _PADW), jnp.int32),
            pltpu.SemaphoreType.DMA((2,)),
        ],
    )(xb, rep)
